# baseline (speedup 1.0000x reference)
import functools

import jax
import jax.numpy as jnp
from jax.experimental import pallas as pl

B, S, N = 128, 2048, 64
E = 8
TOPK = 2
BB = 8


def _gating_kernel(x_ref, ws_ref, bs_ref, wg_ref, bg_ref, gates_ref, load_ref):
    i = pl.program_id(0)

    xb = x_ref[...]
    ws = ws_ref[...]
    wg = wg_ref[...]

    x_lin = jax.lax.dot_general(
        xb.reshape(BB * S, N), ws,
        (((1,), (0,)), ((), ())),
        preferred_element_type=jnp.float32,
    ).reshape(BB, S) + bs_ref[0]

    logits = jax.lax.dot_general(
        x_lin, wg,
        (((1,), (0,)), ((), ())),
        preferred_element_type=jnp.float32,
    ) + bg_ref[...]

    idx = jax.lax.broadcasted_iota(jnp.int32, (BB, E), 1)
    m1 = jnp.max(logits, axis=1, keepdims=True)
    i1 = jnp.min(jnp.where(logits == m1, idx, E), axis=1, keepdims=True)
    masked = jnp.where(idx == i1, -jnp.inf, logits)
    m2 = jnp.max(masked, axis=1, keepdims=True)
    i2 = jnp.min(jnp.where(masked == m2, idx, E), axis=1, keepdims=True)

    t = jnp.exp(m2 - m1)
    denom = 1.0 + t
    g1 = 1.0 / denom
    g2 = t / denom

    gates = jnp.where(idx == i1, g1, jnp.where(idx == i2, g2, 0.0))
    gates_ref[...] = gates

    contrib = (gates > 0.0).astype(jnp.int32)
    partial = jnp.sum(contrib, axis=0, keepdims=True)

    @pl.when(i == 0)
    def _init():
        load_ref[...] = partial

    @pl.when(i != 0)
    def _acc():
        load_ref[...] += partial


@jax.jit
def kernel(x, W_start, b_start, W_gate, b_gate):
    grid = (B // BB,)
    gates, load = pl.pallas_call(
        _gating_kernel,
        grid=grid,
        in_specs=[
            pl.BlockSpec((BB, S, N), lambda i: (i, 0, 0)),
            pl.BlockSpec((N, 1), lambda i: (0, 0)),
            pl.BlockSpec((1,), lambda i: (0,)),
            pl.BlockSpec((S, E), lambda i: (0, 0)),
            pl.BlockSpec((E,), lambda i: (0,)),
        ],
        out_specs=[
            pl.BlockSpec((BB, E), lambda i: (i, 0)),
            pl.BlockSpec((1, E), lambda i: (0, 0)),
        ],
        out_shape=[
            jax.ShapeDtypeStruct((B, E), jnp.float32),
            jax.ShapeDtypeStruct((1, E), jnp.int32),
        ],
    )(x, W_start, b_start, W_gate, b_gate)
    return gates, load.reshape(E)
